# baseline (speedup 1.0000x reference)
import functools

import jax
import jax.numpy as jnp
from jax.experimental import pallas as pl
from jax.experimental.pallas import tpu as pltpu

_B, _S, _H = 1, 2048, 768
_E, _K, _I = 8, 2, 2048
_TB = 256
_NTB = _S // _TB
_NEG = -1e30


def _moe_body(x_ref, wg_ref, wgate_ref, wup_ref, wdown_ref,
              wsg_ref, wsu_ref, wsd_ref, out_ref, acc_ref, w_ref):
    e = pl.program_id(0)
    tb = pl.program_id(1)
    xf = x_ref[...]
    xb = xf.astype(jnp.bfloat16)

    @pl.when(e == 0)
    def _router_and_shared():
        logits = jax.lax.dot_general(
            xf, wg_ref[...], (((1,), (1,)), ((), ())),
            preferred_element_type=jnp.float32)
        ii = jax.lax.broadcasted_iota(jnp.int32, (_TB, _E), 1)
        m0 = jnp.max(logits, axis=1, keepdims=True)
        i0 = jnp.min(jnp.where(logits == m0, ii, _E), axis=1, keepdims=True)
        lm = jnp.where(ii == i0, _NEG, logits)
        m1 = jnp.max(lm, axis=1, keepdims=True)
        i1 = jnp.min(jnp.where(lm == m1, ii, _E), axis=1, keepdims=True)
        g0 = 1.0 / (1.0 + jnp.exp(m1 - m0))
        g1 = 1.0 - g0
        wdense = jnp.where(ii == i0, g0, 0.0) + jnp.where(ii == i1, g1, 0.0)
        w_ref[pl.ds(tb * _TB, _TB), :] = wdense

        sg = jax.lax.dot_general(xb, wsg_ref[...], (((1,), (1,)), ((), ())),
                                 preferred_element_type=jnp.float32)
        su = jax.lax.dot_general(xb, wsu_ref[...], (((1,), (1,)), ((), ())),
                                 preferred_element_type=jnp.float32)
        sh = (jax.nn.silu(sg) * su).astype(jnp.bfloat16)
        shared = jax.lax.dot_general(sh, wsd_ref[...], (((1,), (1,)), ((), ())),
                                     preferred_element_type=jnp.float32)
        acc_ref[pl.ds(tb * _TB, _TB), :] = shared

    wge = wgate_ref[0]
    wue = wup_ref[0]
    wde = wdown_ref[0]
    g = jax.lax.dot_general(xb, wge, (((1,), (1,)), ((), ())),
                            preferred_element_type=jnp.float32)
    u = jax.lax.dot_general(xb, wue, (((1,), (1,)), ((), ())),
                            preferred_element_type=jnp.float32)
    h = (jax.nn.silu(g) * u).astype(jnp.bfloat16)
    y = jax.lax.dot_general(h, wde, (((1,), (1,)), ((), ())),
                            preferred_element_type=jnp.float32)

    ii = jax.lax.broadcasted_iota(jnp.int32, (_TB, _E), 1)
    wcol = jnp.sum(jnp.where(ii == e, w_ref[pl.ds(tb * _TB, _TB), :], 0.0),
                   axis=1, keepdims=True)
    acc = acc_ref[pl.ds(tb * _TB, _TB), :] + wcol * y
    acc_ref[pl.ds(tb * _TB, _TB), :] = acc

    @pl.when(e == _E - 1)
    def _write_out():
        out_ref[...] = acc


def _moe_call(x, wg, wgate, wup, wdown, wsg, wsu, wsd):
    return pl.pallas_call(
        _moe_body,
        grid=(_E, _NTB),
        in_specs=[
            pl.BlockSpec((_TB, _H), lambda e, tb: (tb, 0)),
            pl.BlockSpec((_E, _H), lambda e, tb: (0, 0)),
            pl.BlockSpec((1, _I, _H), lambda e, tb: (e, 0, 0)),
            pl.BlockSpec((1, _I, _H), lambda e, tb: (e, 0, 0)),
            pl.BlockSpec((1, _H, _I), lambda e, tb: (e, 0, 0)),
            pl.BlockSpec((_I, _H), lambda e, tb: (0, 0)),
            pl.BlockSpec((_I, _H), lambda e, tb: (0, 0)),
            pl.BlockSpec((_H, _I), lambda e, tb: (0, 0)),
        ],
        out_specs=pl.BlockSpec((_TB, _H), lambda e, tb: (tb, 0)),
        out_shape=jax.ShapeDtypeStruct((_S, _H), jnp.float32),
        scratch_shapes=[
            pltpu.VMEM((_S, _H), jnp.float32),
            pltpu.VMEM((_S, _E), jnp.float32),
        ],
        compiler_params=pltpu.CompilerParams(
            dimension_semantics=("arbitrary", "arbitrary")),
    )(x, wg, wgate, wup, wdown, wsg, wsu, wsd)


@jax.jit
def kernel(hidden_states, Wg, W_gate, W_up, W_down, Ws_gate, Ws_up, Ws_down):
    b, s, h = hidden_states.shape
    x = hidden_states.reshape(s, h)
    out = _moe_call(
        x, Wg,
        W_gate.astype(jnp.bfloat16), W_up.astype(jnp.bfloat16),
        W_down.astype(jnp.bfloat16),
        Ws_gate.astype(jnp.bfloat16), Ws_up.astype(jnp.bfloat16),
        Ws_down.astype(jnp.bfloat16),
    )
    return out.reshape(b, s, h)


# speedup vs baseline: 1.0466x; 1.0466x over previous
import jax
import jax.numpy as jnp
from jax.experimental import pallas as pl
from jax.experimental.pallas import tpu as pltpu

_B, _S, _H = 1, 2048, 768
_E, _K, _I = 8, 2, 2048
_TB = 256
_NTB = _S // _TB
_IB = 2
_IC = _I // _IB
_NEG = -1e30


def _moe_body(x_ref, wg_ref, wgate_ref, wup_ref, wdown_ref,
              wsg_ref, wsu_ref, wsd_ref, out_ref, acc_ref, w_ref):
    e = pl.program_id(0)
    ib = pl.program_id(1)
    tb = pl.program_id(2)
    xf = x_ref[...]

    @pl.when((e == 0) & (ib == 0))
    def _router_and_shared():
        logits = jax.lax.dot_general(
            xf, wg_ref[...], (((1,), (1,)), ((), ())),
            preferred_element_type=jnp.float32)
        ii = jax.lax.broadcasted_iota(jnp.int32, (_TB, _E), 1)
        m0 = jnp.max(logits, axis=1, keepdims=True)
        i0 = jnp.min(jnp.where(logits == m0, ii, _E), axis=1, keepdims=True)
        lm = jnp.where(ii == i0, _NEG, logits)
        m1 = jnp.max(lm, axis=1, keepdims=True)
        i1 = jnp.min(jnp.where(lm == m1, ii, _E), axis=1, keepdims=True)
        g0 = 1.0 / (1.0 + jnp.exp(m1 - m0))
        g1 = 1.0 - g0
        wdense = jnp.where(ii == i0, g0, 0.0) + jnp.where(ii == i1, g1, 0.0)
        w_ref[pl.ds(tb * _TB, _TB), :] = wdense

        sg = jax.lax.dot_general(xf, wsg_ref[...], (((1,), (1,)), ((), ())),
                                 preferred_element_type=jnp.float32)
        su = jax.lax.dot_general(xf, wsu_ref[...], (((1,), (1,)), ((), ())),
                                 preferred_element_type=jnp.float32)
        sh = jax.nn.silu(sg) * su
        shared = jax.lax.dot_general(sh, wsd_ref[...], (((1,), (1,)), ((), ())),
                                     preferred_element_type=jnp.float32)
        acc_ref[pl.ds(tb * _TB, _TB), :] = shared

    wge = wgate_ref[0]
    wue = wup_ref[0]
    wde = wdown_ref[0]
    g = jax.lax.dot_general(xf, wge, (((1,), (1,)), ((), ())),
                            preferred_element_type=jnp.float32)
    u = jax.lax.dot_general(xf, wue, (((1,), (1,)), ((), ())),
                            preferred_element_type=jnp.float32)
    h = jax.nn.silu(g) * u
    y = jax.lax.dot_general(h, wde, (((1,), (1,)), ((), ())),
                            preferred_element_type=jnp.float32)

    ii = jax.lax.broadcasted_iota(jnp.int32, (_TB, _E), 1)
    wcol = jnp.sum(jnp.where(ii == e, w_ref[pl.ds(tb * _TB, _TB), :], 0.0),
                   axis=1, keepdims=True)
    acc = acc_ref[pl.ds(tb * _TB, _TB), :] + wcol * y
    acc_ref[pl.ds(tb * _TB, _TB), :] = acc

    @pl.when((e == _E - 1) & (ib == _IB - 1))
    def _write_out():
        out_ref[...] = acc


def _moe_call(x, wg, wgate, wup, wdown, wsg, wsu, wsd):
    return pl.pallas_call(
        _moe_body,
        grid=(_E, _IB, _NTB),
        in_specs=[
            pl.BlockSpec((_TB, _H), lambda e, ib, tb: (tb, 0)),
            pl.BlockSpec((_E, _H), lambda e, ib, tb: (0, 0)),
            pl.BlockSpec((1, _IC, _H), lambda e, ib, tb: (e, ib, 0)),
            pl.BlockSpec((1, _IC, _H), lambda e, ib, tb: (e, ib, 0)),
            pl.BlockSpec((1, _H, _IC), lambda e, ib, tb: (e, 0, ib)),
            pl.BlockSpec((_I, _H), lambda e, ib, tb: (0, 0)),
            pl.BlockSpec((_I, _H), lambda e, ib, tb: (0, 0)),
            pl.BlockSpec((_H, _I), lambda e, ib, tb: (0, 0)),
        ],
        out_specs=pl.BlockSpec((_TB, _H), lambda e, ib, tb: (tb, 0)),
        out_shape=jax.ShapeDtypeStruct((_S, _H), jnp.float32),
        scratch_shapes=[
            pltpu.VMEM((_S, _H), jnp.float32),
            pltpu.VMEM((_S, _E), jnp.float32),
        ],
        compiler_params=pltpu.CompilerParams(
            dimension_semantics=("arbitrary", "arbitrary", "arbitrary")),
    )(x, wg, wgate, wup, wdown, wsg, wsu, wsd)


@jax.jit
def kernel(hidden_states, Wg, W_gate, W_up, W_down, Ws_gate, Ws_up, Ws_down):
    b, s, h = hidden_states.shape
    x = hidden_states.reshape(s, h)
    out = _moe_call(x, Wg, W_gate, W_up, W_down, Ws_gate, Ws_up, Ws_down)
    return out.reshape(b, s, h)


# speedup vs baseline: 1.6579x; 1.5842x over previous
import jax
import jax.numpy as jnp
from jax.experimental import pallas as pl
from jax.experimental.pallas import tpu as pltpu

_B, _S, _H = 1, 2048, 768
_E, _K, _I = 8, 2, 2048
_BLK = 256
_NB = 24
_ROWS = _NB * _BLK
_CH = 512
_NEG = -1e30
_TB = 256
_NTB = _S // _TB


def _router_body(x_ref, wg_ref, posw_ref, gw_ref, meta_ref):
    x = x_ref[...]
    logits = jax.lax.dot_general(x, wg_ref[...], (((1,), (1,)), ((), ())),
                                 preferred_element_type=jnp.float32)
    ii = jax.lax.broadcasted_iota(jnp.int32, (_S, _E), 1)
    m0 = jnp.max(logits, axis=1, keepdims=True)
    i0 = jnp.min(jnp.where(logits == m0, ii, _E), axis=1, keepdims=True)
    lm = jnp.where(ii == i0, _NEG, logits)
    m1 = jnp.max(lm, axis=1, keepdims=True)
    i1 = jnp.min(jnp.where(lm == m1, ii, _E), axis=1, keepdims=True)
    g0 = 1.0 / (1.0 + jnp.exp(m1 - m0))
    g1 = 1.0 - g0

    oh0 = (ii == i0).astype(jnp.float32)
    oh1 = (ii == i1).astype(jnp.float32)

    lr = jax.lax.broadcasted_iota(jnp.int32, (_CH, _CH), 0)
    lc = jax.lax.broadcasted_iota(jnp.int32, (_CH, _CH), 1)
    ltri = (lc < lr).astype(jnp.float32)
    carry = jnp.zeros((1, _E), jnp.float32)
    ranks = []
    for oh in (oh0, oh1):
        for c in range(_S // _CH):
            blk = oh[c * _CH:(c + 1) * _CH, :]
            local = jax.lax.dot_general(
                ltri, blk, (((1,), (0,)), ((), ())),
                preferred_element_type=jnp.float32) + carry
            ranks.append(jnp.sum(local * blk, axis=1, keepdims=True))
            carry = carry + jnp.sum(blk, axis=0, keepdims=True)
    counts = carry

    nblk = jnp.floor((counts + (_BLK - 1)) / _BLK)
    er = jax.lax.broadcasted_iota(jnp.int32, (_E, _E), 0)
    ec = jax.lax.broadcasted_iota(jnp.int32, (_E, _E), 1)
    upper = (er < ec).astype(jnp.float32)
    off = _BLK * jax.lax.dot_general(nblk, upper, (((1,), (0,)), ((), ())),
                                     preferred_element_type=jnp.float32)

    rank0 = jnp.concatenate(ranks[:_S // _CH], axis=0)
    rank1 = jnp.concatenate(ranks[_S // _CH:], axis=0)
    pos0 = jnp.sum(oh0 * off, axis=1, keepdims=True) + rank0
    pos1 = jnp.sum(oh1 * off, axis=1, keepdims=True) + rank1
    ci = jax.lax.broadcasted_iota(jnp.int32, (_S, _E), 1)
    posw_ref[...] = jnp.where(
        ci == 0, pos0, jnp.where(ci == 1, pos1, 0.0)).astype(jnp.int32)
    gw_ref[...] = jnp.where(ci == 0, g0, jnp.where(ci == 1, g1, 0.0))

    nb_used = jnp.sum(nblk, axis=1, keepdims=True)
    bi = jax.lax.broadcasted_iota(jnp.int32, (128, _E), 0).astype(jnp.float32)
    row = jnp.minimum(bi, nb_used - 1.0) * _BLK
    offb = off * jnp.ones((128, _E), jnp.float32)
    eob = jnp.sum((row >= offb).astype(jnp.float32), axis=1,
                  keepdims=True) - 1.0
    mc = jax.lax.broadcasted_iota(jnp.int32, (128, _E), 1)
    meta_ref[...] = jnp.where(
        mc == 0, eob, jnp.where(mc == 1, nb_used, 0.0)).astype(jnp.int32)


def _router_call(x, wg):
    return pl.pallas_call(
        _router_body,
        in_specs=[
            pl.BlockSpec((_S, _H), lambda: (0, 0)),
            pl.BlockSpec((_E, _H), lambda: (0, 0)),
        ],
        out_specs=[
            pl.BlockSpec((_S, _E), lambda: (0, 0)),
            pl.BlockSpec((_S, _E), lambda: (0, 0)),
            pl.BlockSpec((128, _E), lambda: (0, 0)),
        ],
        out_shape=[
            jax.ShapeDtypeStruct((_S, _E), jnp.int32),
            jax.ShapeDtypeStruct((_S, _E), jnp.float32),
            jax.ShapeDtypeStruct((128, _E), jnp.int32),
        ],
    )(x, wg)


def _ffn_body(m_ref, posw_ref, x_ref, wgate_ref, wup_ref, wdown_ref, ys_ref):
    b = pl.program_id(0)
    nb = m_ref[0, 1]

    @pl.when(b < nb)
    def _compute():
        p0 = posw_ref[:, 0:1]
        p1 = posw_ref[:, 1:2]
        rr = jax.lax.broadcasted_iota(jnp.int32, (_S, _BLK), 1) + b * _BLK
        m2 = ((rr == p0) | (rr == p1)).astype(jnp.float32)
        xs = jax.lax.dot_general(m2, x_ref[...], (((0,), (0,)), ((), ())),
                                 preferred_element_type=jnp.float32)
        wge = wgate_ref[0]
        wue = wup_ref[0]
        wde = wdown_ref[0]
        g = jax.lax.dot_general(xs, wge, (((1,), (1,)), ((), ())),
                                preferred_element_type=jnp.float32)
        u = jax.lax.dot_general(xs, wue, (((1,), (1,)), ((), ())),
                                preferred_element_type=jnp.float32)
        h = jax.nn.silu(g) * u
        y = jax.lax.dot_general(h, wde, (((1,), (1,)), ((), ())),
                                preferred_element_type=jnp.float32)
        ys_ref[...] = y.astype(jnp.bfloat16)

    @pl.when(b >= nb)
    def _zero():
        ys_ref[...] = jnp.zeros((_BLK, _H), jnp.bfloat16)


def _ffn_call(meta, posw, x, wgate, wup, wdown):
    grid_spec = pltpu.PrefetchScalarGridSpec(
        num_scalar_prefetch=1,
        grid=(_NB,),
        in_specs=[
            pl.BlockSpec((_S, _E), lambda b, m: (0, 0)),
            pl.BlockSpec((_S, _H), lambda b, m: (0, 0)),
            pl.BlockSpec((1, _I, _H), lambda b, m: (m[b, 0], 0, 0)),
            pl.BlockSpec((1, _I, _H), lambda b, m: (m[b, 0], 0, 0)),
            pl.BlockSpec((1, _H, _I), lambda b, m: (m[b, 0], 0, 0)),
        ],
        out_specs=pl.BlockSpec((_BLK, _H), lambda b, m: (b, 0)),
    )
    return pl.pallas_call(
        _ffn_body,
        grid_spec=grid_spec,
        out_shape=jax.ShapeDtypeStruct((_ROWS, _H), jnp.bfloat16),
        compiler_params=pltpu.CompilerParams(
            dimension_semantics=("arbitrary",)),
    )(meta, posw, x, wgate, wup, wdown)


def _combine_body(x_ref, wsg_ref, wsu_ref, wsd_ref, posw_ref, gw_ref,
                  ys_ref, out_ref):
    xb = x_ref[...]
    sg = jax.lax.dot_general(xb, wsg_ref[...], (((1,), (1,)), ((), ())),
                             preferred_element_type=jnp.float32)
    su = jax.lax.dot_general(xb, wsu_ref[...], (((1,), (1,)), ((), ())),
                             preferred_element_type=jnp.float32)
    sh = jax.nn.silu(sg) * su
    shared = jax.lax.dot_general(sh, wsd_ref[...], (((1,), (1,)), ((), ())),
                                 preferred_element_type=jnp.float32)

    p0 = posw_ref[:, 0:1]
    p1 = posw_ref[:, 1:2]
    g0 = gw_ref[:, 0:1]
    g1 = gw_ref[:, 1:2]
    jj = jax.lax.broadcasted_iota(jnp.int32, (_TB, _ROWS), 1)
    gmat = (jnp.where(jj == p0, g0, 0.0)
            + jnp.where(jj == p1, g1, 0.0)).astype(jnp.bfloat16)
    yc = jax.lax.dot_general(gmat, ys_ref[...], (((1,), (0,)), ((), ())),
                             preferred_element_type=jnp.float32)
    out_ref[...] = shared + yc


def _combine_call(x, wsg, wsu, wsd, posw, gw, ys):
    return pl.pallas_call(
        _combine_body,
        grid=(_NTB,),
        in_specs=[
            pl.BlockSpec((_TB, _H), lambda tb: (tb, 0)),
            pl.BlockSpec((_I, _H), lambda tb: (0, 0)),
            pl.BlockSpec((_I, _H), lambda tb: (0, 0)),
            pl.BlockSpec((_H, _I), lambda tb: (0, 0)),
            pl.BlockSpec((_TB, _E), lambda tb: (tb, 0)),
            pl.BlockSpec((_TB, _E), lambda tb: (tb, 0)),
            pl.BlockSpec((_ROWS, _H), lambda tb: (0, 0)),
        ],
        out_specs=pl.BlockSpec((_TB, _H), lambda tb: (tb, 0)),
        out_shape=jax.ShapeDtypeStruct((_S, _H), jnp.float32),
        compiler_params=pltpu.CompilerParams(
            dimension_semantics=("arbitrary",)),
    )(x, wsg, wsu, wsd, posw, gw, ys)


@jax.jit
def kernel(hidden_states, Wg, W_gate, W_up, W_down, Ws_gate, Ws_up, Ws_down):
    b, s, h = hidden_states.shape
    x = hidden_states.reshape(s, h)
    posw, gw, meta = _router_call(x, Wg)
    ys = _ffn_call(meta, posw, x, W_gate, W_up, W_down)
    out = _combine_call(x, Ws_gate, Ws_up, Ws_down, posw, gw, ys)
    return out.reshape(b, s, h)
